# baseline (speedup 1.0000x reference)
import numpy as np
import jax
import jax.numpy as jnp
from jax.experimental import pallas as pl
from jax.experimental.pallas import tpu as pltpu

LENS_L = 7
_R = LENS_L // 2
H = 384
W = 384
WP = 512
RB = 128
RBW = RB + 16
NRB = H // RB
TOP = 8
HS = TOP + H + 8
EPS = 1e-8

_cls = {}
for _u in range(-_R, _R + 1):
    for _v in range(-_R, _R + 1):
        _dsq = _u * _u + _v * _v
        if np.float32(np.sqrt(_dsq)) <= _R + 1e-6:
            _cls.setdefault(_dsq, []).append((_u, _v))
CLASSES = [(np.float32(np.sqrt(d)), len(t), t) for d, t in sorted(_cls.items())]

VPAT = {}
for _ci, (_dist, _cnt, _taps) in enumerate(CLASSES):
    for (_u, _v) in _taps:
        if _v >= 0:
            VPAT.setdefault(_v, []).append((_u, _ci))
for _v in VPAT:
    VPAT[_v].sort()
assert sum(len(p) * (2 if v > 0 else 1) for v, p in VPAT.items()) == 29


def _render_kernel(lens_ref, x_ref, f_ref, out_ref, win_ref, tq_ref, sh_ref):
    b = pl.program_id(0)
    s = pl.program_id(1)
    le = lens_ref[b]
    nch = x_ref.shape[1]
    n_layer = nch // 5

    @pl.when(s < NRB)
    def _():
        for ch in range(nch):
            src = x_ref[0, ch] - f_ref[0, ch]
            left = jnp.broadcast_to(src[:, 0:1], (RB, _R))
            right = jnp.broadcast_to(src[:, W - 1:W], (RB, WP - W - _R))
            win_ref[ch, pl.ds(TOP + s * RB, RB), :] = jnp.concatenate(
                [left, src, right], axis=1)

    @pl.when(s == 0)
    def _():
        for ch in range(nch):
            top = x_ref[0, ch, 0:1, :] - f_ref[0, ch, 0:1, :]
            row = jnp.concatenate(
                [jnp.broadcast_to(top[:, 0:1], (1, _R)), top,
                 jnp.broadcast_to(top[:, W - 1:W], (1, WP - W - _R))], axis=1)
            win_ref[ch, TOP - _R:TOP, :] = jnp.broadcast_to(row, (_R, WP))

    @pl.when(s == NRB - 1)
    def _():
        for ch in range(nch):
            bot = x_ref[0, ch, RB - 1:RB, :] - f_ref[0, ch, RB - 1:RB, :]
            row = jnp.concatenate(
                [jnp.broadcast_to(bot[:, 0:1], (1, _R)), bot,
                 jnp.broadcast_to(bot[:, W - 1:W], (1, WP - W - _R))], axis=1)
            win_ref[ch, TOP + H:TOP + H + _R, :] = jnp.broadcast_to(
                row, (_R, WP))

    @pl.when(s > 0)
    def _():
        row0 = (s - 1) * RB
        voff = TOP - _R
        blur_rgb = None
        trans = None
        for li in range(n_layer):
            rgb = [win_ref[5 * li + c, pl.ds(row0, RBW), :] for c in range(3)]
            a = win_ref[5 * li + 3, pl.ds(row0, RBW), :]
            d = win_ref[5 * li + 4, pl.ds(row0, RBW), :]
            t = jnp.abs(d * le) + 0.5
            norm = jnp.full_like(t, np.float32(CLASSES[0][1]))
            for dist, count, _taps in CLASSES[1:]:
                norm = norm + jnp.where(t >= dist, np.float32(count), 0.0)
            inv = 1.0 / (norm + EPS)

            wa = a * inv
            tq_ref[0] = t
            tq_ref[1] = rgb[0] * wa
            tq_ref[2] = rgb[1] * wa
            tq_ref[3] = rgb[2] * wa
            tq_ref[4] = wa

            for uo in range(LENS_L):
                sh_ref[uo] = tq_ref[:, pl.ds(voff + uo, RB), :]

            S = {}
            for av, pat in VPAT.items():
                acc_v = None
                for (u, ci) in pat:
                    uo = _R + u
                    qblk = sh_ref[uo, 1:5]
                    if ci == 0:
                        term = qblk
                    else:
                        m = sh_ref[uo, 0] >= CLASSES[ci][0]
                        term = jnp.where(m[None, :, :], qblk, 0.0)
                    acc_v = term if acc_v is None else acc_v + term
                S[av] = acc_v

            acc = None
            for v in range(-_R, _R + 1):
                term = jax.lax.slice(S[abs(v)], (0, 0, _R + v),
                                     (4, RB, _R + v + W))
                acc = term if acc is None else acc + term

            ow = acc[3]
            occu = jnp.clip(ow, 0.0, 1.0)
            scale = occu / (ow + EPS)
            layer_rgb = acc[0:3] * scale[None]
            if li == 0:
                blur_rgb = layer_rgb
                trans = 1.0 - occu
            else:
                blur_rgb = blur_rgb + layer_rgb * trans[None]
                trans = trans * (1.0 - occu)

        out_ref[0] = blur_rgb


def kernel(rgbad_layers, lens_effect, focal):
    B, C5, _, _ = rgbad_layers.shape
    le = lens_effect.reshape(B)

    def in_idx(b, s):
        return (b, 0, jnp.minimum(s, NRB - 1), 0)

    out = pl.pallas_call(
        _render_kernel,
        grid=(B, NRB + 1),
        in_specs=[
            pl.BlockSpec(memory_space=pltpu.SMEM),
            pl.BlockSpec((1, C5, RB, W), in_idx),
            pl.BlockSpec((1, C5, RB, W), in_idx),
        ],
        out_specs=pl.BlockSpec((1, 3, RB, W),
                               lambda b, s: (b, 0, jnp.maximum(s - 1, 0), 0)),
        out_shape=jax.ShapeDtypeStruct((B, 3, H, W), jnp.float32),
        scratch_shapes=[pltpu.VMEM((C5, HS, WP), jnp.float32),
                        pltpu.VMEM((5, RBW, WP), jnp.float32),
                        pltpu.VMEM((LENS_L, 5, RB, WP), jnp.float32)],
    )(le, rgbad_layers, focal)
    return out
